# baseline (speedup 1.0000x reference)
import jax
import jax.numpy as jnp
from jax import lax
from jax.experimental import pallas as pl
from jax.experimental.pallas import tpu as pltpu
from jax.experimental.pallas import tpu_sc as plsc

B, F, C, D = 4096, 100, 1000, 64
PADW = 128
NC, NS, L = 2, 16, 16
NW = NC * NS
BPT = B // NW
GRP = 8
GRPS = BPT // GRP
RPI = 10
OFFS = (0, 16, 32, 48, 64, 80, 84)
DOFF = (0, 16, 32, 48)


def _body(emb_hbm, x_hbm, fi_hbm, out_hbm, xb_v, idx_v, fi_v, off_v,
          g_v, s_v, gsem0, gsem1, osem0, osem1):
    wid = lax.axis_index("s") * NC + lax.axis_index("c")
    b_base = wid * BPT
    pltpu.sync_copy(fi_hbm, fi_v)
    for o in OFFS:
        sl = pl.ds(o, L)
        off_v[sl] = fi_v[sl] * C

    gsems = (gsem0, gsem1)
    osems = (osem0, osem1)

    def fire_gather(i, p):
        return pltpu.async_copy(emb_hbm.at[idx_v.at[i]], g_v.at[p], gsems[p])

    def wait_store(p):
        pltpu.make_async_copy(s_v.at[p], out_hbm.at[0], osems[p]).wait()

    def compact(p):
        def rows(r0, carry):
            for rr in range(RPI):
                r = r0 * RPI + rr
                for o in DOFF:
                    sl = pl.ds(o, L)
                    s_v[p, r, sl] = g_v[p, r, sl]
            return carry
        lax.fori_loop(0, F // RPI, rows, 0)

    def group(t, carry):
        b0 = b_base + t * GRP
        pltpu.sync_copy(x_hbm.at[pl.ds(b0, GRP)], xb_v)
        for i in range(GRP):
            for o in OFFS:
                sl = pl.ds(o, L)
                idx_v[i, sl] = xb_v[i, sl] + off_v[sl]

        cps = [None] * GRP
        cps[0] = fire_gather(0, 0)
        for i in range(GRP):
            p = i % 2
            if i + 1 < GRP:
                cps[i + 1] = fire_gather(i + 1, (i + 1) % 2)
            cps[i].wait()
            if i < 2:
                @pl.when(t >= 1)
                def _():
                    wait_store(p)
            else:
                wait_store(p)
            compact(p)
            pltpu.async_copy(s_v.at[p], out_hbm.at[b0 + i], osems[p])
        return carry

    lax.fori_loop(0, GRPS, group, 0)
    wait_store(0)
    wait_store(1)


def _pad_body(e_ref, o_ref):
    o_ref[...] = jnp.concatenate(
        [e_ref[0], jnp.zeros((C, PADW - D), jnp.float32)], axis=-1)


def kernel(x, embedding, feature_idx):
    emb_pad = pl.pallas_call(
        _pad_body,
        grid=(F,),
        in_specs=[pl.BlockSpec((1, C, D), lambda i: (i, 0, 0))],
        out_specs=pl.BlockSpec((C, PADW), lambda i: (i, 0)),
        out_shape=jax.ShapeDtypeStruct((F * C, PADW), jnp.float32),
    )(embedding)
    mesh = plsc.VectorSubcoreMesh(core_axis_name="c", subcore_axis_name="s")
    k = pl.kernel(
        _body,
        mesh=mesh,
        compiler_params=pltpu.CompilerParams(use_tc_tiling_on_sc=True),
        out_type=jax.ShapeDtypeStruct((B, F, D), jnp.float32),
        scratch_types=[
            pltpu.VMEM((GRP, F), jnp.int32),
            pltpu.VMEM((GRP, F), jnp.int32),
            pltpu.VMEM((F,), jnp.int32),
            pltpu.VMEM((F,), jnp.int32),
            pltpu.VMEM((2, F, PADW), jnp.float32),
            pltpu.VMEM((2, F, D), jnp.float32),
            pltpu.SemaphoreType.DMA,
            pltpu.SemaphoreType.DMA,
            pltpu.SemaphoreType.DMA,
            pltpu.SemaphoreType.DMA,
        ],
    )
    return k(emb_pad, x, feature_idx)


# speedup vs baseline: 1.0976x; 1.0976x over previous
import jax
import jax.numpy as jnp
from jax import lax
from jax.experimental import pallas as pl
from jax.experimental.pallas import tpu as pltpu
from jax.experimental.pallas import tpu_sc as plsc

B, F, C, D = 4096, 100, 1000, 64
PADW = 128
NC, NS, L = 2, 16, 16
NW = NC * NS
BPT = B // NW
GRP = 8
GRPS = BPT // GRP
RPI = 10
OFFS = (0, 16, 32, 48, 64, 80, 84)
DOFF = (0, 16, 32, 48)


def _body(emb_hbm, x_hbm, fi_hbm, out_hbm, xb_v, idx_v, fi_v, off_v,
          g_v, s_v, gsem0, gsem1, osem0, osem1):
    wid = lax.axis_index("s") * NC + lax.axis_index("c")
    b_base = wid * BPT
    pltpu.sync_copy(fi_hbm, fi_v)
    for o in OFFS:
        sl = pl.ds(o, L)
        off_v[sl] = fi_v[sl] * C

    gsems = (gsem0, gsem1)
    osems = (osem0, osem1)

    def fire_gather(i, p):
        return pltpu.async_copy(emb_hbm.at[idx_v.at[i]], g_v.at[p], gsems[p])

    def wait_store(p):
        pltpu.make_async_copy(s_v.at[p], out_hbm.at[0], osems[p]).wait()

    def compact(p):
        def rows(r0, carry):
            for rr in range(RPI):
                r = r0 * RPI + rr
                for o in DOFF:
                    sl = pl.ds(o, L)
                    s_v[p, r, sl] = g_v[p, r, sl]
            return carry
        lax.fori_loop(0, F // RPI, rows, 0)

    def group(t, carry):
        b0 = b_base + t * GRP
        pltpu.sync_copy(x_hbm.at[pl.ds(b0, GRP)], xb_v)
        for i in range(GRP):
            for o in OFFS:
                sl = pl.ds(o, L)
                idx_v[i, sl] = xb_v[i, sl] + off_v[sl]

        cps = [None] * GRP
        cps[0] = fire_gather(0, 0)
        for i in range(GRP):
            p = i % 2
            if i + 1 < GRP:
                cps[i + 1] = fire_gather(i + 1, (i + 1) % 2)
            cps[i].wait()
            if i < 2:
                @pl.when(t >= 1)
                def _():
                    wait_store(p)
            else:
                wait_store(p)
            compact(p)
            pltpu.async_copy(s_v.at[p], out_hbm.at[b0 + i], osems[p])
        return carry

    lax.fori_loop(0, GRPS, group, 0)
    wait_store(0)
    wait_store(1)


def kernel(x, embedding, feature_idx):
    emb_pad = jnp.concatenate([embedding, embedding], axis=2).reshape(F * C,
                                                                      PADW)
    mesh = plsc.VectorSubcoreMesh(core_axis_name="c", subcore_axis_name="s")
    k = pl.kernel(
        _body,
        mesh=mesh,
        compiler_params=pltpu.CompilerParams(use_tc_tiling_on_sc=True),
        out_type=jax.ShapeDtypeStruct((B, F, D), jnp.float32),
        scratch_types=[
            pltpu.VMEM((GRP, F), jnp.int32),
            pltpu.VMEM((GRP, F), jnp.int32),
            pltpu.VMEM((F,), jnp.int32),
            pltpu.VMEM((F,), jnp.int32),
            pltpu.VMEM((2, F, PADW), jnp.float32),
            pltpu.VMEM((2, F, D), jnp.float32),
            pltpu.SemaphoreType.DMA,
            pltpu.SemaphoreType.DMA,
            pltpu.SemaphoreType.DMA,
            pltpu.SemaphoreType.DMA,
        ],
    )
    return k(emb_pad, x, feature_idx)


# speedup vs baseline: 1.1780x; 1.0733x over previous
import jax
import jax.numpy as jnp
from jax import lax
from jax.experimental import pallas as pl
from jax.experimental.pallas import tpu as pltpu
from jax.experimental.pallas import tpu_sc as plsc

B, F, C, D = 4096, 100, 1000, 64
PADW = 128
NC, NS, L = 2, 16, 16
NW = NC * NS
BPT = B // NW
GRP = 16
GRPS = BPT // GRP
GB = 4
SB = 2
RPI = 10
OFFS = (0, 16, 32, 48, 64, 80, 84)
DOFF = (0, 16, 32, 48)


def _body(emb_hbm, x_hbm, fi_hbm, out_hbm, xb_v, idx_v, fi_v, off_v,
          g_v, s_v, gsem0, gsem1, gsem2, gsem3, osem0, osem1):
    wid = lax.axis_index("s") * NC + lax.axis_index("c")
    b_base = wid * BPT
    pltpu.sync_copy(fi_hbm, fi_v)
    for o in OFFS:
        sl = pl.ds(o, L)
        off_v[sl] = fi_v[sl] * C

    gsems = (gsem0, gsem1, gsem2, gsem3)
    osems = (osem0, osem1)

    def fire_gather(i):
        p = i % GB
        return pltpu.async_copy(emb_hbm.at[idx_v.at[i]], g_v.at[p], gsems[p])

    def wait_store(p):
        pltpu.make_async_copy(s_v.at[p], out_hbm.at[0], osems[p]).wait()

    def compact(gp, sp):
        def rows(r0, carry):
            for rr in range(RPI):
                r = r0 * RPI + rr
                for o in DOFF:
                    sl = pl.ds(o, L)
                    s_v[sp, r, sl] = g_v[gp, r, sl]
            return carry
        lax.fori_loop(0, F // RPI, rows, 0)

    def group(t, carry):
        b0 = b_base + t * GRP
        pltpu.sync_copy(x_hbm.at[pl.ds(b0, GRP)], xb_v)
        for i in range(GRP):
            for o in OFFS:
                sl = pl.ds(o, L)
                idx_v[i, sl] = xb_v[i, sl] + off_v[sl]

        cps = [None] * GRP
        for i in range(GB - 1):
            cps[i] = fire_gather(i)
        for i in range(GRP):
            gp, sp = i % GB, i % SB
            if i + GB - 1 < GRP:
                cps[i + GB - 1] = fire_gather(i + GB - 1)
            cps[i].wait()
            if i < SB:
                @pl.when(t >= 1)
                def _():
                    wait_store(sp)
            else:
                wait_store(sp)
            compact(gp, sp)
            pltpu.async_copy(s_v.at[sp], out_hbm.at[b0 + i], osems[sp])
        return carry

    lax.fori_loop(0, GRPS, group, 0)
    wait_store(0)
    wait_store(1)


def kernel(x, embedding, feature_idx):
    emb_pad = jnp.pad(embedding,
                      ((0, 0), (0, 0), (0, PADW - D))).reshape(F * C, PADW)
    mesh = plsc.VectorSubcoreMesh(core_axis_name="c", subcore_axis_name="s")
    k = pl.kernel(
        _body,
        mesh=mesh,
        compiler_params=pltpu.CompilerParams(use_tc_tiling_on_sc=True),
        out_type=jax.ShapeDtypeStruct((B, F, D), jnp.float32),
        scratch_types=[
            pltpu.VMEM((GRP, F), jnp.int32),
            pltpu.VMEM((GRP, F), jnp.int32),
            pltpu.VMEM((F,), jnp.int32),
            pltpu.VMEM((F,), jnp.int32),
            pltpu.VMEM((GB, F, PADW), jnp.float32),
            pltpu.VMEM((SB, F, D), jnp.float32),
            pltpu.SemaphoreType.DMA,
            pltpu.SemaphoreType.DMA,
            pltpu.SemaphoreType.DMA,
            pltpu.SemaphoreType.DMA,
            pltpu.SemaphoreType.DMA,
            pltpu.SemaphoreType.DMA,
        ],
    )
    return k(emb_pad, x, feature_idx)
